# Initial kernel scaffold; baseline (speedup 1.0000x reference)
import jax
import jax.numpy as jnp
from jax.experimental import pallas as pl


def kernel(edge_index, x, params):
    raise NotImplementedError("write your pallas kernel here")


# speedup vs baseline: 2.2030x; 2.2030x over previous
import jax
import jax.numpy as jnp
from jax import lax
from jax.experimental import pallas as pl

N = 256
E = 16384
EB = 512
GRID = E // EB

_F32 = jnp.float32


def _dot_t(a, w):
    return lax.dot_general(a, w, (((1,), (1,)), ((), ())),
                           preferred_element_type=_F32)


def _mlp4_block(p, refs):
    W1, b1, W2, b2, W3, b3, W4, b4 = refs
    h = jnp.maximum(_dot_t(p, W1[...]) + b1[...], 0.0)
    h = jnp.maximum(_dot_t(h, W2[...]) + b2[...], 0.0)
    h = jnp.maximum(_dot_t(h, W3[...]) + b3[...], 0.0)
    return _dot_t(h, W4[...]) + b4[...]


def _mlp3_vals(a, refs):
    W1, b1, W2, b2, W3, b3 = refs
    h = jnp.maximum(_dot_t(a, W1[...]) + b1[...], 0.0)
    h = jnp.maximum(_dot_t(h, W2[...]) + b2[...], 0.0)
    return _dot_t(h, W3[...]) + b3[...]


def _mega(src_ref, dst_ref, x_ref, xt_ref, *refs):
    params = refs[:32]
    o1, o2, o3, o4 = refs[32:36]
    e = pl.program_id(0)

    ids_s = src_ref[0]
    ids_d = dst_ref[0]
    iota_n = lax.broadcasted_iota(jnp.int32, (N, EB), 0)
    oh_sT = (iota_n == ids_s).astype(_F32)
    oh_dT = (iota_n == ids_d).astype(_F32)

    x = x_ref[...]
    xt = xt_ref[...]

    def gath(ohT, mat):
        return lax.dot_general(ohT, mat, (((0,), (0,)), ((), ())),
                               preferred_element_type=_F32)

    p = gath(oh_dT, x) * gath(oh_sT, x)
    q = gath(oh_dT, xt) * gath(oh_sT, xt)

    m1 = _mlp4_block(p, params[0:8])
    m2 = _mlp4_block(q, params[8:16])
    m3 = _mlp4_block(p, params[16:24])
    m4 = _mlp4_block(q, params[24:32])

    s1 = jnp.dot(oh_dT, m1, preferred_element_type=_F32)
    s2 = jnp.dot(oh_dT, m2, preferred_element_type=_F32)
    s3 = jnp.dot(oh_sT, m3, preferred_element_type=_F32)
    s4 = jnp.dot(oh_sT, m4, preferred_element_type=_F32)

    @pl.when(e == 0)
    def _init():
        o1[...] = s1
        o2[...] = s2
        o3[...] = s3
        o4[...] = s4

    @pl.when(e != 0)
    def _acc():
        o1[...] += s1
        o2[...] += s2
        o3[...] += s3
        o4[...] += s4


def _finish(a1, a2, a3, a4, *refs):
    fwd = refs[:6]
    bwd = refs[6:12]
    sW1, sb1, sW2, sb2, sW3, sb3 = refs[12:18]
    out = refs[18]
    u1 = _mlp3_vals(a1[...], fwd)
    u2 = _mlp3_vals(a2[...], fwd)
    u3 = _mlp3_vals(a3[...], bwd)
    u4 = _mlp3_vals(a4[...], bwd)
    c = jnp.concatenate([u1, u2, u3, u4], axis=1)
    h = jnp.maximum(_dot_t(c, sW1[...]) + sb1[...], 0.0)
    h = jnp.maximum(_dot_t(h, sW2[...]) + sb2[...], 0.0)
    out[...] = _dot_t(h, sW3[...]) + sb3[...]


def _flatten_mlp(plist):
    flat = []
    for (W, b) in plist:
        flat.append(W)
        flat.append(b.reshape(1, -1))
    return flat


def kernel(edge_index, x, params):
    src = edge_index[0].reshape(GRID, 1, EB)
    dst = edge_index[1].reshape(GRID, 1, EB)
    xt = x.T

    wflat = []
    for name in ("fwd_targets", "fwd_sources", "bwd_targets", "bwd_sources"):
        wflat += _flatten_mlp(params[name])

    idx_spec = pl.BlockSpec((1, 1, EB), lambda e: (e, 0, 0))
    const = lambda shape: pl.BlockSpec(shape, lambda e: (0,) * len(shape))

    in_specs = [idx_spec, idx_spec, const((N, N)), const((N, N))]
    in_specs += [const(w.shape) for w in wflat]

    aggs = pl.pallas_call(
        _mega,
        grid=(GRID,),
        in_specs=in_specs,
        out_specs=[const((N, N))] * 4,
        out_shape=[jax.ShapeDtypeStruct((N, N), _F32)] * 4,
    )(src, dst, x, xt, *wflat)

    s_params = list(params["mlp_2s"])
    W3, b3 = s_params[2]
    W3p = jnp.zeros((128, W3.shape[1]), _F32).at[0].set(W3[0])
    b3p = jnp.zeros((128,), _F32).at[0].set(b3[0])
    s_params[2] = (W3p, b3p)

    fflat = (_flatten_mlp(params["fwd_mlp2"]) +
             _flatten_mlp(params["bwd_mlp2"]) +
             _flatten_mlp(s_params))

    out = pl.pallas_call(
        _finish,
        out_shape=jax.ShapeDtypeStruct((N, 128), _F32),
    )(*aggs, *fflat)
    return out[:, :1]


# speedup vs baseline: 2.4046x; 1.0915x over previous
import jax
import jax.numpy as jnp
from jax import lax
from jax.experimental import pallas as pl

N = 256
E = 16384
EB = 1024
GRID = E // EB

_F32 = jnp.float32


def _dot_t(a, w):
    return lax.dot_general(a, w, (((1,), (1,)), ((), ())),
                           preferred_element_type=_F32)


def _mlp4_block(p, refs):
    W1, b1, W2, b2, W3, b3, W4, b4 = refs
    h = jnp.maximum(_dot_t(p, W1[...]) + b1[...], 0.0)
    h = jnp.maximum(_dot_t(h, W2[...]) + b2[...], 0.0)
    h = jnp.maximum(_dot_t(h, W3[...]) + b3[...], 0.0)
    return _dot_t(h, W4[...]) + b4[...]


def _mlp3_vals(a, refs):
    W1, b1, W2, b2, W3, b3 = refs
    h = jnp.maximum(_dot_t(a, W1[...]) + b1[...], 0.0)
    h = jnp.maximum(_dot_t(h, W2[...]) + b2[...], 0.0)
    return _dot_t(h, W3[...]) + b3[...]


def _mega(src_ref, dst_ref, x_ref, xt_ref, *refs):
    params = refs[:32]
    o1, o2, o3, o4 = refs[32:36]
    e = pl.program_id(0)

    ids_s = src_ref[0]
    ids_d = dst_ref[0]
    iota_n = lax.broadcasted_iota(jnp.int32, (N, EB), 0)
    oh_sT = (iota_n == ids_s).astype(_F32)
    oh_dT = (iota_n == ids_d).astype(_F32)

    x = x_ref[...]
    xt = xt_ref[...]

    def gath(ohT, mat):
        return lax.dot_general(ohT, mat, (((0,), (0,)), ((), ())),
                               preferred_element_type=_F32)

    p = gath(oh_dT, x) * gath(oh_sT, x)
    q = gath(oh_dT, xt) * gath(oh_sT, xt)

    m1 = _mlp4_block(p, params[0:8])
    m2 = _mlp4_block(q, params[8:16])
    m3 = _mlp4_block(p, params[16:24])
    m4 = _mlp4_block(q, params[24:32])

    s1 = jnp.dot(oh_dT, m1, preferred_element_type=_F32)
    s2 = jnp.dot(oh_dT, m2, preferred_element_type=_F32)
    s3 = jnp.dot(oh_sT, m3, preferred_element_type=_F32)
    s4 = jnp.dot(oh_sT, m4, preferred_element_type=_F32)

    @pl.when(e == 0)
    def _init():
        o1[...] = s1
        o2[...] = s2
        o3[...] = s3
        o4[...] = s4

    @pl.when(e != 0)
    def _acc():
        o1[...] += s1
        o2[...] += s2
        o3[...] += s3
        o4[...] += s4


def _finish(a1, a2, a3, a4, *refs):
    fwd = refs[:6]
    bwd = refs[6:12]
    sW1, sb1, sW2, sb2, sW3, sb3 = refs[12:18]
    out = refs[18]
    u1 = _mlp3_vals(a1[...], fwd)
    u2 = _mlp3_vals(a2[...], fwd)
    u3 = _mlp3_vals(a3[...], bwd)
    u4 = _mlp3_vals(a4[...], bwd)
    c = jnp.concatenate([u1, u2, u3, u4], axis=1)
    h = jnp.maximum(_dot_t(c, sW1[...]) + sb1[...], 0.0)
    h = jnp.maximum(_dot_t(h, sW2[...]) + sb2[...], 0.0)
    out[...] = _dot_t(h, sW3[...]) + sb3[...]


def _flatten_mlp(plist):
    flat = []
    for (W, b) in plist:
        flat.append(W)
        flat.append(b.reshape(1, -1))
    return flat


def kernel(edge_index, x, params):
    src = edge_index[0].reshape(GRID, 1, EB)
    dst = edge_index[1].reshape(GRID, 1, EB)
    xt = x.T

    wflat = []
    for name in ("fwd_targets", "fwd_sources", "bwd_targets", "bwd_sources"):
        wflat += _flatten_mlp(params[name])

    idx_spec = pl.BlockSpec((1, 1, EB), lambda e: (e, 0, 0))
    const = lambda shape: pl.BlockSpec(shape, lambda e: (0,) * len(shape))

    in_specs = [idx_spec, idx_spec, const((N, N)), const((N, N))]
    in_specs += [const(w.shape) for w in wflat]

    aggs = pl.pallas_call(
        _mega,
        grid=(GRID,),
        in_specs=in_specs,
        out_specs=[const((N, N))] * 4,
        out_shape=[jax.ShapeDtypeStruct((N, N), _F32)] * 4,
    )(src, dst, x, xt, *wflat)

    s_params = list(params["mlp_2s"])
    W3, b3 = s_params[2]
    W3p = jnp.zeros((128, W3.shape[1]), _F32).at[0].set(W3[0])
    b3p = jnp.zeros((128,), _F32).at[0].set(b3[0])
    s_params[2] = (W3p, b3p)

    fflat = (_flatten_mlp(params["fwd_mlp2"]) +
             _flatten_mlp(params["bwd_mlp2"]) +
             _flatten_mlp(s_params))

    out = pl.pallas_call(
        _finish,
        out_shape=jax.ShapeDtypeStruct((N, 128), _F32),
    )(*aggs, *fflat)
    return out[:, :1]


# speedup vs baseline: 2.4663x; 1.0257x over previous
import jax
import jax.numpy as jnp
from jax import lax
from jax.experimental import pallas as pl

N = 256
E = 16384
EB = 2048
GRID = E // EB

_F32 = jnp.float32


def _dot_t(a, w):
    return lax.dot_general(a, w, (((1,), (1,)), ((), ())),
                           preferred_element_type=_F32)


def _mlp4_block(p, refs):
    W1, b1, W2, b2, W3, b3, W4, b4 = refs
    h = jnp.maximum(_dot_t(p, W1[...]) + b1[...], 0.0)
    h = jnp.maximum(_dot_t(h, W2[...]) + b2[...], 0.0)
    h = jnp.maximum(_dot_t(h, W3[...]) + b3[...], 0.0)
    return _dot_t(h, W4[...]) + b4[...]


def _mlp3_vals(a, refs):
    W1, b1, W2, b2, W3, b3 = refs
    h = jnp.maximum(_dot_t(a, W1[...]) + b1[...], 0.0)
    h = jnp.maximum(_dot_t(h, W2[...]) + b2[...], 0.0)
    return _dot_t(h, W3[...]) + b3[...]


def _mega(src_ref, dst_ref, x_ref, xt_ref, *refs):
    params = refs[:32]
    o1, o2, o3, o4 = refs[32:36]
    e = pl.program_id(0)

    ids_s = src_ref[0]
    ids_d = dst_ref[0]
    iota_n = lax.broadcasted_iota(jnp.int32, (N, EB), 0)
    oh_sT = (iota_n == ids_s).astype(_F32)
    oh_dT = (iota_n == ids_d).astype(_F32)

    x = x_ref[...]
    xt = xt_ref[...]

    def gath(ohT, mat):
        return lax.dot_general(ohT, mat, (((0,), (0,)), ((), ())),
                               preferred_element_type=_F32)

    p = gath(oh_dT, x) * gath(oh_sT, x)
    q = gath(oh_dT, xt) * gath(oh_sT, xt)

    m1 = _mlp4_block(p, params[0:8])
    m2 = _mlp4_block(q, params[8:16])
    m3 = _mlp4_block(p, params[16:24])
    m4 = _mlp4_block(q, params[24:32])

    s1 = jnp.dot(oh_dT, m1, preferred_element_type=_F32)
    s2 = jnp.dot(oh_dT, m2, preferred_element_type=_F32)
    s3 = jnp.dot(oh_sT, m3, preferred_element_type=_F32)
    s4 = jnp.dot(oh_sT, m4, preferred_element_type=_F32)

    @pl.when(e == 0)
    def _init():
        o1[...] = s1
        o2[...] = s2
        o3[...] = s3
        o4[...] = s4

    @pl.when(e != 0)
    def _acc():
        o1[...] += s1
        o2[...] += s2
        o3[...] += s3
        o4[...] += s4


def _finish(a1, a2, a3, a4, *refs):
    fwd = refs[:6]
    bwd = refs[6:12]
    sW1, sb1, sW2, sb2, sW3, sb3 = refs[12:18]
    out = refs[18]
    u1 = _mlp3_vals(a1[...], fwd)
    u2 = _mlp3_vals(a2[...], fwd)
    u3 = _mlp3_vals(a3[...], bwd)
    u4 = _mlp3_vals(a4[...], bwd)
    c = jnp.concatenate([u1, u2, u3, u4], axis=1)
    h = jnp.maximum(_dot_t(c, sW1[...]) + sb1[...], 0.0)
    h = jnp.maximum(_dot_t(h, sW2[...]) + sb2[...], 0.0)
    out[...] = _dot_t(h, sW3[...]) + sb3[...]


def _flatten_mlp(plist):
    flat = []
    for (W, b) in plist:
        flat.append(W)
        flat.append(b.reshape(1, -1))
    return flat


def kernel(edge_index, x, params):
    src = edge_index[0].reshape(GRID, 1, EB)
    dst = edge_index[1].reshape(GRID, 1, EB)
    xt = x.T

    wflat = []
    for name in ("fwd_targets", "fwd_sources", "bwd_targets", "bwd_sources"):
        wflat += _flatten_mlp(params[name])

    idx_spec = pl.BlockSpec((1, 1, EB), lambda e: (e, 0, 0))
    const = lambda shape: pl.BlockSpec(shape, lambda e: (0,) * len(shape))

    in_specs = [idx_spec, idx_spec, const((N, N)), const((N, N))]
    in_specs += [const(w.shape) for w in wflat]

    aggs = pl.pallas_call(
        _mega,
        grid=(GRID,),
        in_specs=in_specs,
        out_specs=[const((N, N))] * 4,
        out_shape=[jax.ShapeDtypeStruct((N, N), _F32)] * 4,
    )(src, dst, x, xt, *wflat)

    s_params = list(params["mlp_2s"])
    W3, b3 = s_params[2]
    W3p = jnp.zeros((128, W3.shape[1]), _F32).at[0].set(W3[0])
    b3p = jnp.zeros((128,), _F32).at[0].set(b3[0])
    s_params[2] = (W3p, b3p)

    fflat = (_flatten_mlp(params["fwd_mlp2"]) +
             _flatten_mlp(params["bwd_mlp2"]) +
             _flatten_mlp(s_params))

    out = pl.pallas_call(
        _finish,
        out_shape=jax.ShapeDtypeStruct((N, 128), _F32),
    )(*aggs, *fflat)
    return out[:, :1]
